# baseline (speedup 1.0000x reference)
import functools

import jax
import jax.numpy as jnp
from jax import lax
from jax.experimental import pallas as pl
from jax.experimental.pallas import tpu as pltpu
from jax.experimental.pallas import tpu_sc as plsc

N = 10000
E = 320000
D = 128
K = 3

NC = 2
NS = 16
NW = NC * NS
NP = 10240
ROWS_PER_TILE = NP // NS
EP = 327680
EDGES_PER_TILE = EP // NW
CHUNK = 128
NCHUNK = EDGES_PER_TILE // CHUNK
NBUF = 2

_mesh = plsc.VectorSubcoreMesh(core_axis_name="c", subcore_axis_name="s")


def _fill(ref, nrows, value):
    v = jnp.full((16,), value, jnp.float32)
    for r in range(nrows):
        for j in range(D // 16):
            ref[r, pl.ds(j * 16, 16)] = v


def _zero_acc_slice(rows0, acc, row0):
    _fill(rows0, CHUNK, 0.0)
    for t in range(ROWS_PER_TILE // CHUNK):
        pltpu.sync_copy(rows0, acc.at[pl.ds(row0 + t * CHUNK, CHUNK)])


def _scaled_flush(rows0, acc, sbuf, row0, cid, out_hbm):
    def scale_rows(t, carry):
        pltpu.sync_copy(acc.at[pl.ds(row0 + t * CHUNK, CHUNK)], rows0)

        def srow16(r16, c2):
            sv = sbuf[pl.ds(t * CHUNK + r16 * 16, 16)]
            for i in range(16):
                svec = jnp.full((16,), sv[i], jnp.float32)
                for j in range(D // 16):
                    sl = pl.ds(j * 16, 16)
                    rows0[r16 * 16 + i, sl] = rows0[r16 * 16 + i, sl] * svec
            return c2

        lax.fori_loop(0, CHUNK // 16, srow16, 0)

        pltpu.sync_copy(rows0,
                        out_hbm.at[cid, pl.ds(row0 + t * CHUNK, CHUNK)])
        return carry

    lax.fori_loop(0, ROWS_PER_TILE // CHUNK, scale_rows, 0)


def _make_prop(dual):
    n_z = 2 if dual else 1

    @functools.partial(
        pl.kernel,
        mesh=_mesh,
        out_type=jax.ShapeDtypeStruct((NC, NP, D), jnp.float32),
        scratch_types=[
            pltpu.VMEM((EDGES_PER_TILE,), jnp.int32),
            pltpu.VMEM((NBUF, CHUNK), jnp.int32),
            pltpu.VMEM((NBUF, CHUNK, D), jnp.float32),
            pltpu.VMEM((ROWS_PER_TILE,), jnp.float32),
            pltpu.VMEM_SHARED((NP, D), jnp.float32),
            pltpu.SemaphoreType.DMA,
            pltpu.SemaphoreType.DMA,
            pltpu.SemaphoreType.DMA((NBUF,)),
            pltpu.SemaphoreType.DMA((NBUF,)),
            pltpu.SemaphoreType.DMA((NBUF,)),
        ],
    )
    def prop(z_hbm, src_hbm, dst_hbm, sneg_hbm, out_hbm,
             sidx, didx, rows, sbuf, acc, semi, semsc, semd, semg, sems):
        z_views = ([z_hbm.at[0], z_hbm.at[1]] if dual else [z_hbm])
        cid = lax.axis_index("c")
        sid = lax.axis_index("s")
        wid = sid * NC + cid
        ebase = wid * EDGES_PER_TILE
        row0 = sid * ROWS_PER_TILE

        ia = pltpu.async_copy(src_hbm.at[pl.ds(ebase, EDGES_PER_TILE)],
                              sidx, semi)
        ib = pltpu.async_copy(sneg_hbm.at[pl.ds(row0, ROWS_PER_TILE)],
                              sbuf, semsc)
        _zero_acc_slice(rows.at[0], acc, row0)
        ia.wait()
        plsc.subcore_barrier()

        def load_didx(ci, b):
            off = pl.multiple_of(ebase + ci * CHUNK, CHUNK)
            pltpu.async_copy(dst_hbm.at[pl.ds(off, CHUNK)], didx.at[b],
                             semd.at[b])

        def wait_didx(b):
            pltpu.make_async_copy(dst_hbm.at[pl.ds(0, CHUNK)], didx.at[b],
                                  semd.at[b]).wait()

        def gather(ci, b):
            soff = pl.multiple_of(ci * CHUNK, CHUNK)
            zsrc = z_views[b % n_z]
            pltpu.async_copy(zsrc.at[sidx.at[pl.ds(soff, CHUNK)]],
                             rows.at[b], semg.at[b])

        def wait_gather(b):
            pltpu.make_async_copy(z_views[0].at[sidx.at[pl.ds(0, CHUNK)]],
                                  rows.at[b], semg.at[b]).wait()

        def scatter(b):
            pltpu.async_copy(rows.at[b], acc.at[didx.at[b]], sems.at[b],
                             add=True)

        def wait_scatter(b):
            pltpu.make_async_copy(rows.at[b], acc.at[didx.at[0]],
                                  sems.at[b]).wait()

        def cix(step, b):
            return step if dual else step * NBUF + b

        nsteps = NCHUNK if dual else NCHUNK // NBUF
        for b in range(NBUF):
            load_didx(cix(0, b), b)
            gather(cix(0, b), b)

        def body(g, carry):
            for b in range(NBUF):
                wait_gather(b)
                wait_didx(b)
                scatter(b)
            for b in range(NBUF):
                wait_scatter(b)
                load_didx(cix(g + 1, b), b)
                gather(cix(g + 1, b), b)
            return carry

        lax.fori_loop(0, nsteps - 1, body, 0)
        for b in range(NBUF):
            wait_gather(b)
            wait_didx(b)
            scatter(b)
        for b in range(NBUF):
            wait_scatter(b)
        plsc.subcore_barrier()

        ib.wait()
        _scaled_flush(rows.at[0], acc, sbuf, row0, cid, out_hbm)

    return prop


_sc_prop1 = _make_prop(dual=False)
_sc_prop2 = _make_prop(dual=True)


@functools.partial(
    pl.kernel,
    mesh=_mesh,
    out_type=jax.ShapeDtypeStruct((NC, NP, D), jnp.float32),
    scratch_types=[
        pltpu.VMEM((NBUF, CHUNK), jnp.int32),
        pltpu.VMEM((CHUNK, D), jnp.float32),
        pltpu.VMEM_SHARED((NP, D), jnp.float32),
        pltpu.SemaphoreType.DMA((NBUF,)),
        pltpu.SemaphoreType.DMA((NBUF,)),
    ],
)
def _sc_degree(src_hbm, out_hbm, sidx, ones, acc, semi, sems):
    cid = lax.axis_index("c")
    sid = lax.axis_index("s")
    wid = sid * NC + cid
    ebase = wid * EDGES_PER_TILE

    _fill(ones, CHUNK, 0.0)
    row0 = sid * ROWS_PER_TILE
    for t in range(ROWS_PER_TILE // CHUNK):
        pltpu.sync_copy(ones, acc.at[pl.ds(row0 + t * CHUNK, CHUNK)])
    _fill(ones, CHUNK, 1.0)
    plsc.subcore_barrier()

    def load_sidx(ci, b):
        off = pl.multiple_of(ebase + ci * CHUNK, CHUNK)
        pltpu.async_copy(src_hbm.at[pl.ds(off, CHUNK)], sidx.at[b],
                         semi.at[b])

    def wait_sidx(b):
        pltpu.make_async_copy(src_hbm.at[pl.ds(0, CHUNK)], sidx.at[b],
                              semi.at[b]).wait()

    def scatter(b):
        pltpu.async_copy(ones, acc.at[sidx.at[b]], sems.at[b], add=True)

    def wait_scatter(b):
        pltpu.make_async_copy(ones, acc.at[sidx.at[0]], sems.at[b]).wait()

    for b in range(NBUF):
        load_sidx(b, b)

    def body(g, carry):
        base = g * NBUF
        for b in range(NBUF):
            wait_sidx(b)
            scatter(b)
        for b in range(NBUF):
            wait_scatter(b)
            load_sidx(base + NBUF + b, b)
        return carry

    lax.fori_loop(0, NCHUNK // NBUF - 1, body, 0)
    for b in range(NBUF):
        wait_sidx(b)
        scatter(b)
    for b in range(NBUF):
        wait_scatter(b)
    plsc.subcore_barrier()

    pltpu.sync_copy(acc.at[pl.ds(row0, ROWS_PER_TILE)],
                    out_hbm.at[cid, pl.ds(row0, ROWS_PER_TILE)])


RB = 512
GRID = NP // RB


def _tc_prep_body(deg_ref, x_ref, dinv_ref, rinv_ref, sneg_ref, z0_ref):
    deg = deg_ref[0, :, 0:1] + deg_ref[1, :, 0:1]
    pos = deg > 0.0
    safe = jnp.maximum(deg, 1.0)
    dinv = jnp.where(pos, lax.rsqrt(safe), 0.0)
    dinv_ref[...] = dinv
    rinv_ref[...] = jnp.where(pos, jnp.sqrt(safe), 0.0)
    sneg_ref[...] = -(dinv * dinv)
    z0_ref[...] = x_ref[...] * dinv


def _tc_prep(deg_parts, x):
    return pl.pallas_call(
        _tc_prep_body,
        grid=(GRID,),
        in_specs=[
            pl.BlockSpec((NC, RB, D), lambda i: (0, i, 0)),
            pl.BlockSpec((RB, D), lambda i: (i, 0)),
        ],
        out_specs=[
            pl.BlockSpec((RB, 1), lambda i: (i, 0)),
            pl.BlockSpec((RB, 1), lambda i: (i, 0)),
            pl.BlockSpec((RB, 1), lambda i: (i, 0)),
            pl.BlockSpec((RB, D), lambda i: (i, 0)),
        ],
        out_shape=[
            jax.ShapeDtypeStruct((NP, 1), jnp.float32),
            jax.ShapeDtypeStruct((NP, 1), jnp.float32),
            jax.ShapeDtypeStruct((NP, 1), jnp.float32),
            jax.ShapeDtypeStruct((NP, D), jnp.float32),
        ],
    )(deg_parts, x)


def _tc_finish_body(h_ref, za_ref, zb_ref, dinv_ref,
                    rinv_ref, w_ref, b_ref, g_ref, be_ref, h2_ref, z2_ref,
                    *, last):
    h = h_ref[...]
    rinv = rinv_ref[...]
    tx1 = (za_ref[0] + za_ref[1]) * rinv
    tx2 = 2.0 * (zb_ref[0] + zb_ref[1]) * rinv - h
    out = jnp.dot(h, w_ref[0], preferred_element_type=jnp.float32)
    out += jnp.dot(tx1, w_ref[1], preferred_element_type=jnp.float32)
    out += jnp.dot(tx2, w_ref[2], preferred_element_type=jnp.float32)
    out += b_ref[...]
    if last:
        h2_ref[...] = out
        z2_ref[...] = out
    else:
        mu = jnp.mean(out, axis=-1, keepdims=True)
        var = jnp.mean((out - mu) * (out - mu), axis=-1, keepdims=True)
        ln = (out - mu) * lax.rsqrt(var + 1e-5) * g_ref[...] + be_ref[...]
        act = jnp.where(ln > 0.0, ln,
                        jnp.exp(jnp.minimum(ln, 0.0)) - 1.0)
        h2_ref[...] = act
        z2_ref[...] = act * dinv_ref[...]


def _tc_finish(h, za, zb, dinv, rinv, W, b, g, be, last):
    return pl.pallas_call(
        functools.partial(_tc_finish_body, last=last),
        grid=(GRID,),
        in_specs=[
            pl.BlockSpec((RB, D), lambda i: (i, 0)),
            pl.BlockSpec((NC, RB, D), lambda i: (0, i, 0)),
            pl.BlockSpec((NC, RB, D), lambda i: (0, i, 0)),
            pl.BlockSpec((RB, 1), lambda i: (i, 0)),
            pl.BlockSpec((RB, 1), lambda i: (i, 0)),
            pl.BlockSpec((K, D, D), lambda i: (0, 0, 0)),
            pl.BlockSpec((1, D), lambda i: (0, 0)),
            pl.BlockSpec((1, D), lambda i: (0, 0)),
            pl.BlockSpec((1, D), lambda i: (0, 0)),
        ],
        out_specs=[
            pl.BlockSpec((RB, D), lambda i: (i, 0)),
            pl.BlockSpec((RB, D), lambda i: (i, 0)),
        ],
        out_shape=[
            jax.ShapeDtypeStruct((NP, D), jnp.float32),
            jax.ShapeDtypeStruct((NP, D), jnp.float32),
        ],
    )(h, za, zb, dinv, rinv, W, b, g, be)


def kernel(x, edge_index, W0, b0, W1, b1, W2, b2, W3, b3,
           g0, be0, g1, be1, g2, be2):
    src = edge_index[0]
    dst = edge_index[1]
    pad = EP - E
    src_p = jnp.concatenate([src, jnp.full((pad,), N, jnp.int32)])
    dst_p = jnp.concatenate([dst, jnp.full((pad,), N, jnp.int32)])
    x_p = jnp.zeros((NP, D), x.dtype).at[:N].set(x)

    deg_parts = _sc_degree(src_p)
    dinv, rinv, sneg, z = _tc_prep(deg_parts, x_p)
    sneg1 = sneg.reshape(NP)

    h = x_p
    params = [(W0, b0, g0, be0), (W1, b1, g1, be1),
              (W2, b2, g2, be2), (W3, b3, None, None)]
    for li, (W, b, g, be) in enumerate(params):
        last = li == 3
        za = _sc_prop1(z, src_p, dst_p, sneg1)
        zb = _sc_prop2(za, src_p, dst_p, sneg1)
        if last:
            g = jnp.ones((D,), jnp.float32)
            be = jnp.zeros((D,), jnp.float32)
        h, z = _tc_finish(h, za, zb, dinv, rinv, W,
                          b.reshape(1, D), g.reshape(1, D),
                          be.reshape(1, D), last)
    return h[:N]


# speedup vs baseline: 1.1379x; 1.1379x over previous
import functools

import jax
import jax.numpy as jnp
from jax import lax
from jax.experimental import pallas as pl
from jax.experimental.pallas import tpu as pltpu
from jax.experimental.pallas import tpu_sc as plsc

N = 10000
E = 320000
D = 128
K = 3

NC = 2
NS = 16
NW = NC * NS
NP = 10240
ROWS_PER_TILE = NP // NS
EP = 327680
EDGES_PER_TILE = EP // NW
CHUNK = 128
NCHUNK = EDGES_PER_TILE // CHUNK
NBUF = 2
ZROWS = 64

_mesh = plsc.VectorSubcoreMesh(core_axis_name="c", subcore_axis_name="s")


def _fill(ref, nrows, value):
    v = jnp.full((16,), value, jnp.float32)
    for r in range(nrows):
        for j in range(D // 16):
            ref[r, pl.ds(j * 16, 16)] = v


@functools.partial(
    pl.kernel,
    mesh=_mesh,
    out_type=jax.ShapeDtypeStruct((NC, NP, D), jnp.float32),
    scratch_types=[
        pltpu.VMEM((EDGES_PER_TILE,), jnp.int32),
        pltpu.VMEM((NBUF, CHUNK), jnp.int32),
        pltpu.VMEM((NBUF, CHUNK, D), jnp.float32),
        pltpu.VMEM_SHARED((NP, D), jnp.float32),
        pltpu.SemaphoreType.DMA,
        pltpu.SemaphoreType.DMA((NBUF,)),
        pltpu.SemaphoreType.DMA((NBUF,)),
        pltpu.SemaphoreType.DMA((NBUF,)),
    ],
)
def _sc_prop(z_hbm, src_hbm, dst_hbm, out_hbm,
             sidx, didx, rows, acc, semi, semd, semg, sems):
    cid = lax.axis_index("c")
    sid = lax.axis_index("s")
    wid = sid * NC + cid
    ebase = wid * EDGES_PER_TILE

    ia = pltpu.async_copy(src_hbm.at[pl.ds(ebase, EDGES_PER_TILE)], sidx, semi)
    _fill(rows.at[0], CHUNK, 0.0)
    row0 = sid * ROWS_PER_TILE
    for t in range(ROWS_PER_TILE // CHUNK):
        pltpu.sync_copy(rows.at[0], acc.at[pl.ds(row0 + t * CHUNK, CHUNK)])
    ia.wait()
    plsc.subcore_barrier()

    def load_didx(ci, b):
        off = pl.multiple_of(ebase + ci * CHUNK, CHUNK)
        pltpu.async_copy(dst_hbm.at[pl.ds(off, CHUNK)], didx.at[b],
                         semd.at[b])

    def wait_didx(b):
        pltpu.make_async_copy(dst_hbm.at[pl.ds(0, CHUNK)], didx.at[b],
                              semd.at[b]).wait()

    def gather(ci, b):
        soff = pl.multiple_of(ci * CHUNK, CHUNK)
        pltpu.async_copy(z_hbm.at[sidx.at[pl.ds(soff, CHUNK)]], rows.at[b],
                         semg.at[b])

    def wait_gather(b):
        pltpu.make_async_copy(z_hbm.at[sidx.at[pl.ds(0, CHUNK)]], rows.at[b],
                              semg.at[b]).wait()

    def scatter(b):
        pltpu.async_copy(rows.at[b], acc.at[didx.at[b]], sems.at[b],
                         add=True)

    def wait_scatter(b):
        pltpu.make_async_copy(rows.at[b], acc.at[didx.at[0]],
                              sems.at[b]).wait()

    for b in range(NBUF):
        load_didx(b, b)
        gather(b, b)

    def body(g, carry):
        base = g * NBUF
        for b in range(NBUF):
            wait_gather(b)
            wait_didx(b)
            scatter(b)
        for b in range(NBUF):
            nci = base + NBUF + b
            wait_scatter(b)
            load_didx(nci, b)
            gather(nci, b)
        return carry

    lax.fori_loop(0, NCHUNK // NBUF - 1, body, 0)
    for b in range(NBUF):
        wait_gather(b)
        wait_didx(b)
        scatter(b)
    for b in range(NBUF):
        wait_scatter(b)
    plsc.subcore_barrier()

    pltpu.sync_copy(acc.at[pl.ds(row0, ROWS_PER_TILE)],
                    out_hbm.at[cid, pl.ds(row0, ROWS_PER_TILE)])


@functools.partial(
    pl.kernel,
    mesh=_mesh,
    out_type=jax.ShapeDtypeStruct((NC, NP, D), jnp.float32),
    scratch_types=[
        pltpu.VMEM((NBUF, CHUNK), jnp.int32),
        pltpu.VMEM((CHUNK, D), jnp.float32),
        pltpu.VMEM_SHARED((NP, D), jnp.float32),
        pltpu.SemaphoreType.DMA((NBUF,)),
        pltpu.SemaphoreType.DMA((NBUF,)),
    ],
)
def _sc_degree(src_hbm, out_hbm, sidx, ones, acc, semi, sems):
    cid = lax.axis_index("c")
    sid = lax.axis_index("s")
    wid = sid * NC + cid
    ebase = wid * EDGES_PER_TILE

    _fill(ones, CHUNK, 0.0)
    row0 = sid * ROWS_PER_TILE
    for t in range(ROWS_PER_TILE // CHUNK):
        pltpu.sync_copy(ones, acc.at[pl.ds(row0 + t * CHUNK, CHUNK)])
    _fill(ones, CHUNK, 1.0)
    plsc.subcore_barrier()

    def load_sidx(ci, b):
        off = pl.multiple_of(ebase + ci * CHUNK, CHUNK)
        pltpu.async_copy(src_hbm.at[pl.ds(off, CHUNK)], sidx.at[b],
                         semi.at[b])

    def wait_sidx(b):
        pltpu.make_async_copy(src_hbm.at[pl.ds(0, CHUNK)], sidx.at[b],
                              semi.at[b]).wait()

    def scatter(b):
        pltpu.async_copy(ones, acc.at[sidx.at[b]], sems.at[b], add=True)

    def wait_scatter(b):
        pltpu.make_async_copy(ones, acc.at[sidx.at[0]], sems.at[b]).wait()

    for b in range(NBUF):
        load_sidx(b, b)

    def body(g, carry):
        base = g * NBUF
        for b in range(NBUF):
            wait_sidx(b)
            scatter(b)
        for b in range(NBUF):
            wait_scatter(b)
            load_sidx(base + NBUF + b, b)
        return carry

    lax.fori_loop(0, NCHUNK // NBUF - 1, body, 0)
    for b in range(NBUF):
        wait_sidx(b)
        scatter(b)
    for b in range(NBUF):
        wait_scatter(b)
    plsc.subcore_barrier()

    pltpu.sync_copy(acc.at[pl.ds(row0, ROWS_PER_TILE)],
                    out_hbm.at[cid, pl.ds(row0, ROWS_PER_TILE)])


RB = 512
GRID = NP // RB


def _tc_prep_body(deg_ref, x_ref, dinv_ref, z0_ref):
    deg = deg_ref[0, :, 0:1] + deg_ref[1, :, 0:1]
    dinv = jnp.where(deg > 0.0, lax.rsqrt(jnp.maximum(deg, 1.0)), 0.0)
    dinv_ref[...] = dinv
    z0_ref[...] = x_ref[...] * dinv


def _tc_prep(deg_parts, x):
    return pl.pallas_call(
        _tc_prep_body,
        grid=(GRID,),
        in_specs=[
            pl.BlockSpec((NC, RB, D), lambda i: (0, i, 0)),
            pl.BlockSpec((RB, D), lambda i: (i, 0)),
        ],
        out_specs=[
            pl.BlockSpec((RB, 1), lambda i: (i, 0)),
            pl.BlockSpec((RB, D), lambda i: (i, 0)),
        ],
        out_shape=[
            jax.ShapeDtypeStruct((NP, 1), jnp.float32),
            jax.ShapeDtypeStruct((NP, D), jnp.float32),
        ],
    )(deg_parts, x)


def _tc_mid_body(s0_ref, dinv_ref, tx1_ref, z1_ref):
    s0 = s0_ref[0] + s0_ref[1]
    dinv = dinv_ref[...]
    tx1 = -dinv * s0
    tx1_ref[...] = tx1
    z1_ref[...] = dinv * tx1


def _tc_mid(s0_parts, dinv):
    return pl.pallas_call(
        _tc_mid_body,
        grid=(GRID,),
        in_specs=[
            pl.BlockSpec((NC, RB, D), lambda i: (0, i, 0)),
            pl.BlockSpec((RB, 1), lambda i: (i, 0)),
        ],
        out_specs=[
            pl.BlockSpec((RB, D), lambda i: (i, 0)),
            pl.BlockSpec((RB, D), lambda i: (i, 0)),
        ],
        out_shape=[
            jax.ShapeDtypeStruct((NP, D), jnp.float32),
            jax.ShapeDtypeStruct((NP, D), jnp.float32),
        ],
    )(s0_parts, dinv)


def _tc_finish_body(h_ref, tx1_ref, s1_ref, dinv_ref, w_ref, b_ref, g_ref,
                    be_ref, h2_ref, z2_ref, *, last):
    h = h_ref[...]
    tx1 = tx1_ref[...]
    dinv = dinv_ref[...]
    tx2 = -2.0 * dinv * (s1_ref[0] + s1_ref[1]) - h
    out = jnp.dot(h, w_ref[0], preferred_element_type=jnp.float32)
    out += jnp.dot(tx1, w_ref[1], preferred_element_type=jnp.float32)
    out += jnp.dot(tx2, w_ref[2], preferred_element_type=jnp.float32)
    out += b_ref[...]
    if last:
        h2_ref[...] = out
        z2_ref[...] = out
    else:
        mu = jnp.mean(out, axis=-1, keepdims=True)
        var = jnp.mean((out - mu) * (out - mu), axis=-1, keepdims=True)
        ln = (out - mu) * lax.rsqrt(var + 1e-5) * g_ref[...] + be_ref[...]
        act = jnp.where(ln > 0.0, ln,
                        jnp.exp(jnp.minimum(ln, 0.0)) - 1.0)
        h2_ref[...] = act
        z2_ref[...] = act * dinv


def _tc_finish(h, tx1, s1_parts, dinv, W, b, g, be, last):
    return pl.pallas_call(
        functools.partial(_tc_finish_body, last=last),
        grid=(GRID,),
        in_specs=[
            pl.BlockSpec((RB, D), lambda i: (i, 0)),
            pl.BlockSpec((RB, D), lambda i: (i, 0)),
            pl.BlockSpec((NC, RB, D), lambda i: (0, i, 0)),
            pl.BlockSpec((RB, 1), lambda i: (i, 0)),
            pl.BlockSpec((K, D, D), lambda i: (0, 0, 0)),
            pl.BlockSpec((1, D), lambda i: (0, 0)),
            pl.BlockSpec((1, D), lambda i: (0, 0)),
            pl.BlockSpec((1, D), lambda i: (0, 0)),
        ],
        out_specs=[
            pl.BlockSpec((RB, D), lambda i: (i, 0)),
            pl.BlockSpec((RB, D), lambda i: (i, 0)),
        ],
        out_shape=[
            jax.ShapeDtypeStruct((NP, D), jnp.float32),
            jax.ShapeDtypeStruct((NP, D), jnp.float32),
        ],
    )(h, tx1, s1_parts, dinv, W, b, g, be)


def kernel(x, edge_index, W0, b0, W1, b1, W2, b2, W3, b3,
           g0, be0, g1, be1, g2, be2):
    src = edge_index[0]
    dst = edge_index[1]
    pad = EP - E
    src_p = jnp.concatenate([src, jnp.full((pad,), N, jnp.int32)])
    dst_p = jnp.concatenate([dst, jnp.full((pad,), N, jnp.int32)])
    x_p = jnp.zeros((NP, D), x.dtype).at[:N].set(x)

    deg_parts = _sc_degree(src_p)
    dinv, z = _tc_prep(deg_parts, x_p)

    h = x_p
    params = [(W0, b0, g0, be0), (W1, b1, g1, be1),
              (W2, b2, g2, be2), (W3, b3, None, None)]
    for li, (W, b, g, be) in enumerate(params):
        last = li == 3
        s0_parts = _sc_prop(z, src_p, dst_p)
        tx1, z1 = _tc_mid(s0_parts, dinv)
        s1_parts = _sc_prop(z1, src_p, dst_p)
        if last:
            g = jnp.ones((D,), jnp.float32)
            be = jnp.zeros((D,), jnp.float32)
        h, z = _tc_finish(h, tx1, s1_parts, dinv, W,
                          b.reshape(1, D), g.reshape(1, D),
                          be.reshape(1, D), last)
    return h[:N]


# speedup vs baseline: 1.1432x; 1.0046x over previous
import functools

import jax
import jax.numpy as jnp
from jax import lax
from jax.experimental import pallas as pl
from jax.experimental.pallas import tpu as pltpu
from jax.experimental.pallas import tpu_sc as plsc

N = 10000
E = 320000
D = 128
K = 3

NC = 2
NS = 16
NW = NC * NS
NP = 10240
ROWS_PER_TILE = NP // NS
EP = 327680
EDGES_PER_TILE = EP // NW
CHUNK = 64
NCHUNK = EDGES_PER_TILE // CHUNK
NBUF = 4
ZROWS = 64

_mesh = plsc.VectorSubcoreMesh(core_axis_name="c", subcore_axis_name="s")


def _fill(ref, nrows, value):
    v = jnp.full((16,), value, jnp.float32)
    for r in range(nrows):
        for j in range(D // 16):
            ref[r, pl.ds(j * 16, 16)] = v


@functools.partial(
    pl.kernel,
    mesh=_mesh,
    out_type=jax.ShapeDtypeStruct((NC, NP, D), jnp.float32),
    scratch_types=[
        pltpu.VMEM((EDGES_PER_TILE,), jnp.int32),
        pltpu.VMEM((NBUF, CHUNK), jnp.int32),
        pltpu.VMEM((NBUF, CHUNK, D), jnp.float32),
        pltpu.VMEM_SHARED((NP, D), jnp.float32),
        pltpu.SemaphoreType.DMA,
        pltpu.SemaphoreType.DMA((NBUF,)),
        pltpu.SemaphoreType.DMA((NBUF,)),
        pltpu.SemaphoreType.DMA((NBUF,)),
    ],
)
def _sc_prop(z_hbm, src_hbm, dst_hbm, out_hbm,
             sidx, didx, rows, acc, semi, semd, semg, sems):
    cid = lax.axis_index("c")
    sid = lax.axis_index("s")
    wid = sid * NC + cid
    ebase = wid * EDGES_PER_TILE

    ia = pltpu.async_copy(src_hbm.at[pl.ds(ebase, EDGES_PER_TILE)], sidx, semi)
    _fill(rows.at[0], CHUNK, 0.0)
    row0 = sid * ROWS_PER_TILE
    for t in range(ROWS_PER_TILE // CHUNK):
        pltpu.sync_copy(rows.at[0], acc.at[pl.ds(row0 + t * CHUNK, CHUNK)])
    ia.wait()
    plsc.subcore_barrier()

    def load_didx(ci, b):
        off = pl.multiple_of(ebase + ci * CHUNK, CHUNK)
        pltpu.async_copy(dst_hbm.at[pl.ds(off, CHUNK)], didx.at[b],
                         semd.at[b])

    def wait_didx(b):
        pltpu.make_async_copy(dst_hbm.at[pl.ds(0, CHUNK)], didx.at[b],
                              semd.at[b]).wait()

    def gather(ci, b):
        soff = pl.multiple_of(ci * CHUNK, CHUNK)
        pltpu.async_copy(z_hbm.at[sidx.at[pl.ds(soff, CHUNK)]], rows.at[b],
                         semg.at[b])

    def wait_gather(b):
        pltpu.make_async_copy(z_hbm.at[sidx.at[pl.ds(0, CHUNK)]], rows.at[b],
                              semg.at[b]).wait()

    def scatter(b):
        pltpu.async_copy(rows.at[b], acc.at[didx.at[b]], sems.at[b],
                         add=True)

    def wait_scatter(b):
        pltpu.make_async_copy(rows.at[b], acc.at[didx.at[0]],
                              sems.at[b]).wait()

    for b in range(NBUF):
        load_didx(b, b)
        gather(b, b)

    def body(g, carry):
        base = g * NBUF
        for b in range(NBUF):
            wait_gather(b)
            wait_didx(b)
            scatter(b)
        for b in range(NBUF):
            nci = base + NBUF + b
            wait_scatter(b)
            load_didx(nci, b)
            gather(nci, b)
        return carry

    lax.fori_loop(0, NCHUNK // NBUF - 1, body, 0)
    for b in range(NBUF):
        wait_gather(b)
        wait_didx(b)
        scatter(b)
    for b in range(NBUF):
        wait_scatter(b)
    plsc.subcore_barrier()

    pltpu.sync_copy(acc.at[pl.ds(row0, ROWS_PER_TILE)],
                    out_hbm.at[cid, pl.ds(row0, ROWS_PER_TILE)])


@functools.partial(
    pl.kernel,
    mesh=_mesh,
    out_type=jax.ShapeDtypeStruct((NC, NP, D), jnp.float32),
    scratch_types=[
        pltpu.VMEM((NBUF, CHUNK), jnp.int32),
        pltpu.VMEM((CHUNK, D), jnp.float32),
        pltpu.VMEM_SHARED((NP, D), jnp.float32),
        pltpu.SemaphoreType.DMA((NBUF,)),
        pltpu.SemaphoreType.DMA((NBUF,)),
    ],
)
def _sc_degree(src_hbm, out_hbm, sidx, ones, acc, semi, sems):
    cid = lax.axis_index("c")
    sid = lax.axis_index("s")
    wid = sid * NC + cid
    ebase = wid * EDGES_PER_TILE

    _fill(ones, CHUNK, 0.0)
    row0 = sid * ROWS_PER_TILE
    for t in range(ROWS_PER_TILE // CHUNK):
        pltpu.sync_copy(ones, acc.at[pl.ds(row0 + t * CHUNK, CHUNK)])
    _fill(ones, CHUNK, 1.0)
    plsc.subcore_barrier()

    def load_sidx(ci, b):
        off = pl.multiple_of(ebase + ci * CHUNK, CHUNK)
        pltpu.async_copy(src_hbm.at[pl.ds(off, CHUNK)], sidx.at[b],
                         semi.at[b])

    def wait_sidx(b):
        pltpu.make_async_copy(src_hbm.at[pl.ds(0, CHUNK)], sidx.at[b],
                              semi.at[b]).wait()

    def scatter(b):
        pltpu.async_copy(ones, acc.at[sidx.at[b]], sems.at[b], add=True)

    def wait_scatter(b):
        pltpu.make_async_copy(ones, acc.at[sidx.at[0]], sems.at[b]).wait()

    for b in range(NBUF):
        load_sidx(b, b)

    def body(g, carry):
        base = g * NBUF
        for b in range(NBUF):
            wait_sidx(b)
            scatter(b)
        for b in range(NBUF):
            wait_scatter(b)
            load_sidx(base + NBUF + b, b)
        return carry

    lax.fori_loop(0, NCHUNK // NBUF - 1, body, 0)
    for b in range(NBUF):
        wait_sidx(b)
        scatter(b)
    for b in range(NBUF):
        wait_scatter(b)
    plsc.subcore_barrier()

    pltpu.sync_copy(acc.at[pl.ds(row0, ROWS_PER_TILE)],
                    out_hbm.at[cid, pl.ds(row0, ROWS_PER_TILE)])


RB = 512
GRID = NP // RB


def _tc_prep_body(deg_ref, x_ref, dinv_ref, z0_ref):
    deg = deg_ref[0, :, 0:1] + deg_ref[1, :, 0:1]
    dinv = jnp.where(deg > 0.0, lax.rsqrt(jnp.maximum(deg, 1.0)), 0.0)
    dinv_ref[...] = dinv
    z0_ref[...] = x_ref[...] * dinv


def _tc_prep(deg_parts, x):
    return pl.pallas_call(
        _tc_prep_body,
        grid=(GRID,),
        in_specs=[
            pl.BlockSpec((NC, RB, D), lambda i: (0, i, 0)),
            pl.BlockSpec((RB, D), lambda i: (i, 0)),
        ],
        out_specs=[
            pl.BlockSpec((RB, 1), lambda i: (i, 0)),
            pl.BlockSpec((RB, D), lambda i: (i, 0)),
        ],
        out_shape=[
            jax.ShapeDtypeStruct((NP, 1), jnp.float32),
            jax.ShapeDtypeStruct((NP, D), jnp.float32),
        ],
    )(deg_parts, x)


def _tc_mid_body(s0_ref, dinv_ref, tx1_ref, z1_ref):
    s0 = s0_ref[0] + s0_ref[1]
    dinv = dinv_ref[...]
    tx1 = -dinv * s0
    tx1_ref[...] = tx1
    z1_ref[...] = dinv * tx1


def _tc_mid(s0_parts, dinv):
    return pl.pallas_call(
        _tc_mid_body,
        grid=(GRID,),
        in_specs=[
            pl.BlockSpec((NC, RB, D), lambda i: (0, i, 0)),
            pl.BlockSpec((RB, 1), lambda i: (i, 0)),
        ],
        out_specs=[
            pl.BlockSpec((RB, D), lambda i: (i, 0)),
            pl.BlockSpec((RB, D), lambda i: (i, 0)),
        ],
        out_shape=[
            jax.ShapeDtypeStruct((NP, D), jnp.float32),
            jax.ShapeDtypeStruct((NP, D), jnp.float32),
        ],
    )(s0_parts, dinv)


def _tc_finish_body(h_ref, tx1_ref, s1_ref, dinv_ref, w_ref, b_ref, g_ref,
                    be_ref, h2_ref, z2_ref, *, last):
    h = h_ref[...]
    tx1 = tx1_ref[...]
    dinv = dinv_ref[...]
    tx2 = -2.0 * dinv * (s1_ref[0] + s1_ref[1]) - h
    out = jnp.dot(h, w_ref[0], preferred_element_type=jnp.float32)
    out += jnp.dot(tx1, w_ref[1], preferred_element_type=jnp.float32)
    out += jnp.dot(tx2, w_ref[2], preferred_element_type=jnp.float32)
    out += b_ref[...]
    if last:
        h2_ref[...] = out
        z2_ref[...] = out
    else:
        mu = jnp.mean(out, axis=-1, keepdims=True)
        var = jnp.mean((out - mu) * (out - mu), axis=-1, keepdims=True)
        ln = (out - mu) * lax.rsqrt(var + 1e-5) * g_ref[...] + be_ref[...]
        act = jnp.where(ln > 0.0, ln,
                        jnp.exp(jnp.minimum(ln, 0.0)) - 1.0)
        h2_ref[...] = act
        z2_ref[...] = act * dinv


def _tc_finish(h, tx1, s1_parts, dinv, W, b, g, be, last):
    return pl.pallas_call(
        functools.partial(_tc_finish_body, last=last),
        grid=(GRID,),
        in_specs=[
            pl.BlockSpec((RB, D), lambda i: (i, 0)),
            pl.BlockSpec((RB, D), lambda i: (i, 0)),
            pl.BlockSpec((NC, RB, D), lambda i: (0, i, 0)),
            pl.BlockSpec((RB, 1), lambda i: (i, 0)),
            pl.BlockSpec((K, D, D), lambda i: (0, 0, 0)),
            pl.BlockSpec((1, D), lambda i: (0, 0)),
            pl.BlockSpec((1, D), lambda i: (0, 0)),
            pl.BlockSpec((1, D), lambda i: (0, 0)),
        ],
        out_specs=[
            pl.BlockSpec((RB, D), lambda i: (i, 0)),
            pl.BlockSpec((RB, D), lambda i: (i, 0)),
        ],
        out_shape=[
            jax.ShapeDtypeStruct((NP, D), jnp.float32),
            jax.ShapeDtypeStruct((NP, D), jnp.float32),
        ],
    )(h, tx1, s1_parts, dinv, W, b, g, be)


def kernel(x, edge_index, W0, b0, W1, b1, W2, b2, W3, b3,
           g0, be0, g1, be1, g2, be2):
    src = edge_index[0]
    dst = edge_index[1]
    pad = EP - E
    src_p = jnp.concatenate([src, jnp.full((pad,), N, jnp.int32)])
    dst_p = jnp.concatenate([dst, jnp.full((pad,), N, jnp.int32)])
    x_p = jnp.zeros((NP, D), x.dtype).at[:N].set(x)

    deg_parts = _sc_degree(src_p)
    dinv, z = _tc_prep(deg_parts, x_p)

    h = x_p
    params = [(W0, b0, g0, be0), (W1, b1, g1, be1),
              (W2, b2, g2, be2), (W3, b3, None, None)]
    for li, (W, b, g, be) in enumerate(params):
        last = li == 3
        s0_parts = _sc_prop(z, src_p, dst_p)
        tx1, z1 = _tc_mid(s0_parts, dinv)
        s1_parts = _sc_prop(z1, src_p, dst_p)
        if last:
            g = jnp.ones((D,), jnp.float32)
            be = jnp.zeros((D,), jnp.float32)
        h, z = _tc_finish(h, tx1, s1_parts, dinv, W,
                          b.reshape(1, D), g.reshape(1, D),
                          be.reshape(1, D), last)
    return h[:N]
